# baseline (speedup 1.0000x reference)
import functools

import jax
import jax.numpy as jnp
from jax import lax
from jax.experimental import pallas as pl
from jax.experimental.pallas import tpu as pltpu
from jax.experimental.pallas import tpu_sc as plsc

_NE = 1024
_D = 64
_DP = 128
_HW = 576
_B = 64
_N = _B * _HW

_NC = 2
_NS = 16
_NW = _NC * _NS
_BPT = _B // _NW


def _vq_body(z_ref, emb_ref, embm2_ref, idx_ref, loss_ref):
    x = z_ref[0]
    emb = emb_ref[...]
    a2 = jnp.sum(x * x, axis=1, keepdims=True)
    b2 = jnp.sum(emb * emb, axis=1)[None, :]
    ab2 = lax.dot_general(x, embm2_ref[...], (((1,), (1,)), ((), ())),
                          preferred_element_type=jnp.float32)
    sq = (a2 + b2) + ab2
    d = jnp.sqrt(jnp.maximum(sq, 0.0))
    dmin = jnp.min(d, axis=1, keepdims=True)
    ji = lax.broadcasted_iota(jnp.int32, (_HW, _NE), 1)
    idx = jnp.min(jnp.where(d == dmin, ji, jnp.int32(2**30)), axis=1)
    idx_ref[0, 0, :] = idx
    part = jnp.sum(dmin * dmin)
    @pl.when(pl.program_id(0) == 0)
    def _():
        loss_ref[0, 0] = 0.0
    loss_ref[0, 0] += part


_sc_mesh = plsc.VectorSubcoreMesh(core_axis_name="c", subcore_axis_name="s")


@functools.partial(
    pl.kernel,
    mesh=_sc_mesh,
    out_type=jax.ShapeDtypeStruct((_B, _HW, _DP), jnp.float32),
    scratch_types=[
        pltpu.VMEM((_HW,), jnp.int32),
        pltpu.VMEM((_HW, _DP), jnp.float32),
        pltpu.SemaphoreType.DMA,
    ],
)
def _gather_rows(idx_hbm, tab_hbm, out_hbm, idx_v, rows_v, sem):
    wid = lax.axis_index("s") * _NC + lax.axis_index("c")
    for bl in range(_BPT):
        b = wid * _BPT + bl
        pltpu.sync_copy(idx_hbm.at[pl.ds(b * _HW, _HW)], idx_v)
        pltpu.async_copy(tab_hbm.at[idx_v], rows_v, sem).wait()
        pltpu.sync_copy(rows_v, out_hbm.at[b])


@jax.jit
def kernel(z, embeddings):
    zf = z.reshape(_B, _HW, _D)
    idx3, loss_acc = pl.pallas_call(
        _vq_body,
        grid=(_B,),
        in_specs=[
            pl.BlockSpec((1, _HW, _D), lambda i: (i, 0, 0)),
            pl.BlockSpec((_NE, _D), lambda i: (0, 0)),
            pl.BlockSpec((_NE, _D), lambda i: (0, 0)),
        ],
        out_specs=[
            pl.BlockSpec((1, 1, _HW), lambda i: (i, 0, 0)),
            pl.BlockSpec(memory_space=pltpu.SMEM, block_shape=(1, 1),
                         index_map=lambda i: (0, 0)),
        ],
        out_shape=[
            jax.ShapeDtypeStruct((_B, 1, _HW), jnp.int32),
            jax.ShapeDtypeStruct((1, 1), jnp.float32),
        ],
        compiler_params=pltpu.CompilerParams(
            dimension_semantics=("arbitrary",)),
    )(zf, embeddings, embeddings * jnp.float32(-2.0))
    tab_pad = jnp.concatenate(
        [embeddings, jnp.zeros((_NE, _DP - _D), jnp.float32)], axis=1)
    qp = _gather_rows(idx3.reshape(_N), tab_pad)
    quantized = qp[:, :, :_D].transpose(0, 2, 1).reshape(_B, _D, 24, 24)
    indices = idx3.reshape(_B, 1, 24, 24)
    loss = (loss_acc[0, 0] / jnp.float32(_N * _D)) * jnp.float32(1.25)
    return quantized, indices, loss


# speedup vs baseline: 1.2211x; 1.2211x over previous
import functools

import jax
import jax.numpy as jnp
from jax import lax
from jax.experimental import pallas as pl
from jax.experimental.pallas import tpu as pltpu
from jax.experimental.pallas import tpu_sc as plsc

_NE = 1024
_D = 64
_DP = 128
_HW = 576
_B = 64
_N = _B * _HW

_NC = 2
_NS = 16
_NW = _NC * _NS
_BPT = _B // _NW
_RG = 4
_RPS = _RG * _HW


def _vq_body(z_ref, emb_ref, embm2_ref, idx_ref, loss_ref):
    x = z_ref[...].reshape(_RPS, _D)
    emb = emb_ref[...]
    a2 = jnp.sum(x * x, axis=1, keepdims=True)
    b2 = jnp.sum(emb * emb, axis=1)[None, :]
    ab2 = lax.dot_general(x, embm2_ref[...], (((1,), (1,)), ((), ())),
                          preferred_element_type=jnp.float32)
    sq = (a2 + b2) + ab2
    d = jnp.sqrt(jnp.maximum(sq, 0.0))
    dmin = jnp.min(d, axis=1, keepdims=True)
    ji = lax.broadcasted_iota(jnp.int32, (_RPS, _NE), 1)
    idx = jnp.min(jnp.where(d == dmin, ji, jnp.int32(2**30)), axis=1)
    idx_ref[0, 0, :] = idx
    part = jnp.sum(dmin * dmin)
    @pl.when(pl.program_id(0) == 0)
    def _():
        loss_ref[0, 0] = 0.0
    loss_ref[0, 0] += part


_sc_mesh = plsc.VectorSubcoreMesh(core_axis_name="c", subcore_axis_name="s")


@functools.partial(
    pl.kernel,
    mesh=_sc_mesh,
    out_type=jax.ShapeDtypeStruct((_B, _HW, _DP), jnp.float32),
    scratch_types=[
        pltpu.VMEM((_HW,), jnp.int32),
        pltpu.VMEM((_HW, _DP), jnp.float32),
        pltpu.SemaphoreType.DMA,
    ],
)
def _gather_rows(idx_hbm, tab_hbm, out_hbm, idx_v, rows_v, sem):
    wid = lax.axis_index("s") * _NC + lax.axis_index("c")
    for bl in range(_BPT):
        b = wid * _BPT + bl
        pltpu.sync_copy(idx_hbm.at[pl.ds(b * _HW, _HW)], idx_v)
        pltpu.async_copy(tab_hbm.at[idx_v], rows_v, sem).wait()
        pltpu.sync_copy(rows_v, out_hbm.at[b])


@jax.jit
def kernel(z, embeddings):
    zf = z.reshape(_B // _RG, _RPS, _D)
    idx3, loss_acc = pl.pallas_call(
        _vq_body,
        grid=(_B // _RG,),
        in_specs=[
            pl.BlockSpec((1, _RPS, _D), lambda i: (i, 0, 0)),
            pl.BlockSpec((_NE, _D), lambda i: (0, 0)),
            pl.BlockSpec((_NE, _D), lambda i: (0, 0)),
        ],
        out_specs=[
            pl.BlockSpec((1, 1, _RPS), lambda i: (i, 0, 0)),
            pl.BlockSpec(memory_space=pltpu.SMEM, block_shape=(1, 1),
                         index_map=lambda i: (0, 0)),
        ],
        out_shape=[
            jax.ShapeDtypeStruct((_B // _RG, 1, _RPS), jnp.int32),
            jax.ShapeDtypeStruct((1, 1), jnp.float32),
        ],
        compiler_params=pltpu.CompilerParams(
            dimension_semantics=("arbitrary",)),
    )(zf, embeddings, embeddings * jnp.float32(-2.0))
    tab_pad = jnp.concatenate(
        [embeddings, jnp.zeros((_NE, _DP - _D), jnp.float32)], axis=1)
    qp = _gather_rows(idx3.reshape(_N), tab_pad)
    quantized = qp[:, :, :_D].transpose(0, 2, 1).reshape(_B, _D, 24, 24)
    indices = idx3.reshape(_B, 1, 24, 24)
    loss = (loss_acc[0, 0] / jnp.float32(_N * _D)) * jnp.float32(1.25)
    return quantized, indices, loss
